# baseline (speedup 1.0000x reference)
import functools

import jax
import jax.numpy as jnp
from jax import lax
from jax.experimental import pallas as pl
from jax.experimental.pallas import tpu as pltpu
from jax.experimental.pallas import tpu_sc as plsc

_HID = 1024
_SRC = 2048
_BATCH = 4
_N = _SRC * _BATCH
_NW = 32
_RPW = _N // _NW
_C = 16
_NCH = _RPW // _C
_PC = _C // _BATCH
_NL = _HID // 16
_EPS = 1e-5
_UNROLL = 8
_NJB = _NL // _UNROLL


def _sc_body(ids_hbm, pids_hbm, wemb_hbm, pemb_hbm, gam_hbm, bet_hbm,
             out_hbm,
             ids_all, pid_all, tok_a, tok_b, pos_a, pos_b,
             out_a, out_b, g_v, b_v, sacc_v, qacc_v, m_v, r_v,
             sem_ta, sem_tb, sem_pa, sem_pb, sem_wa, sem_wb):
    wid = lax.axis_index("s") * 2 + lax.axis_index("c")
    lanes = lax.iota(jnp.int32, 16)
    z = jnp.zeros((16,), jnp.float32)

    pltpu.sync_copy(ids_hbm.at[pl.ds(wid * _NCH, _NCH), :], ids_all)
    pltpu.sync_copy(pids_hbm.at[pl.ds(wid * _NCH, _NCH), :], pid_all)
    pltpu.sync_copy(gam_hbm, g_v)
    pltpu.sync_copy(bet_hbm, b_v)

    def issue_gather(c, tok_v, pos_v, sem_t, sem_p):
        pltpu.async_copy(wemb_hbm.at[ids_all.at[c]], tok_v, sem_t)
        pltpu.async_copy(pemb_hbm.at[pid_all.at[c]], pos_v, sem_p)

    def drain_gather(tok_v, pos_v, sem_t, sem_p):
        pltpu.make_async_copy(wemb_hbm.at[pl.ds(0, _C), :], tok_v,
                              sem_t).wait()
        pltpu.make_async_copy(pemb_hbm.at[pl.ds(0, _PC), :], pos_v,
                              sem_p).wait()

    def drain_writeback(out_v, sem_w):
        pltpu.make_async_copy(out_v, out_hbm.at[pl.ds(0, _C), :],
                              sem_w).wait()

    def compute(c, tok_v, pos_v, out_v, sem_w):
        base = pl.multiple_of(wid * _RPW + c * _C, _C)

        @plsc.parallel_loop(0, _C)
        def pass1_row(i):
            p = i // _BATCH

            @plsc.parallel_loop(0, _NJB, unroll=8, carry=(z, z, z, z))
            def acc(jb, sc):
                s0, s1, q0, q1 = sc
                for u in range(_UNROLL):
                    off = jb * (_UNROLL * 16) + u * 16
                    x = (tok_v[i, pl.ds(off, 16)]
                         + pos_v[p, pl.ds(off, 16)])
                    out_v[i, pl.ds(off, 16)] = x
                    if u % 2 == 0:
                        s0 = s0 + x
                        q0 = q0 + x * x
                    else:
                        s1 = s1 + x
                        q1 = q1 + x * x
                return (s0, s1, q0, q1)

            s0, s1, q0, q1 = acc
            sacc_v[i, :] = s0 + s1
            qacc_v[i, :] = q0 + q1

        t = z
        tq = z
        for k in range(16):
            ck = jnp.full((16,), k, jnp.int32)
            t = t + plsc.load_gather(sacc_v, [lanes, ck])
            tq = tq + plsc.load_gather(qacc_v, [lanes, ck])
        mean16 = t * (1.0 / _HID)
        var16 = tq * (1.0 / _HID) - mean16 * mean16
        xv = var16 + _EPS
        ii = plsc.bitcast(xv, jnp.int32)
        ii = 0x5F3759DF - (ii >> 1)
        y = plsc.bitcast(ii, jnp.float32)
        y = y * (1.5 - 0.5 * xv * y * y)
        y = y * (1.5 - 0.5 * xv * y * y)
        y = y * (1.5 - 0.5 * xv * y * y)
        r_v[pl.ds(8, 16)] = y
        m_v[pl.ds(8, 16)] = mean16 * y

        for rb in range(_C // 4):
            yv = [plsc.load_gather(r_v, [jnp.full((16,), 8 + rb * 4 + i,
                                                  jnp.int32)])
                  for i in range(4)]
            nm = [plsc.load_gather(m_v, [jnp.full((16,), 8 + rb * 4 + i,
                                                  jnp.int32)])
                  for i in range(4)]

            @plsc.parallel_loop(0, _NJB, unroll=8)
            def norm(jb):
                for u in range(_UNROLL):
                    off = jb * (_UNROLL * 16) + u * 16
                    g = g_v[pl.ds(off, 16)]
                    b = b_v[pl.ds(off, 16)]
                    for i in range(4):
                        row = rb * 4 + i
                        x = out_v[row, pl.ds(off, 16)]
                        out_v[row, pl.ds(off, 16)] = (
                            (x * yv[i] - nm[i]) * g + b)
        pltpu.async_copy(out_v, out_hbm.at[pl.ds(base, _C), :], sem_w)

    issue_gather(0, tok_a, pos_a, sem_ta, sem_pa)

    def pair_body(g, carry):
        c0 = g * 2
        issue_gather(c0 + 1, tok_b, pos_b, sem_tb, sem_pb)
        drain_gather(tok_a, pos_a, sem_ta, sem_pa)

        @pl.when(g > 0)
        def _():
            drain_writeback(out_a, sem_wa)

        compute(c0, tok_a, pos_a, out_a, sem_wa)

        @pl.when(g < _NCH // 2 - 1)
        def _():
            issue_gather(c0 + 2, tok_a, pos_a, sem_ta, sem_pa)

        drain_gather(tok_b, pos_b, sem_tb, sem_pb)

        @pl.when(g > 0)
        def _():
            drain_writeback(out_b, sem_wb)

        compute(c0 + 1, tok_b, pos_b, out_b, sem_wb)
        return carry

    lax.fori_loop(0, _NCH // 2, pair_body, 0)
    drain_writeback(out_a, sem_wa)
    drain_writeback(out_b, sem_wb)


_sc_embed = functools.partial(
    pl.kernel,
    mesh=plsc.VectorSubcoreMesh(core_axis_name="c", subcore_axis_name="s"),
    out_type=jax.ShapeDtypeStruct((_N, _HID), jnp.float32),
    compiler_params=pltpu.CompilerParams(needs_layout_passes=False),
    scratch_types=[
        pltpu.VMEM((_NCH, _C), jnp.int32),
        pltpu.VMEM((_NCH, _PC), jnp.int32),
        pltpu.VMEM((_C, _HID), jnp.float32),
        pltpu.VMEM((_C, _HID), jnp.float32),
        pltpu.VMEM((_PC, _HID), jnp.float32),
        pltpu.VMEM((_PC, _HID), jnp.float32),
        pltpu.VMEM((_C, _HID), jnp.float32),
        pltpu.VMEM((_C, _HID), jnp.float32),
        pltpu.VMEM((_HID,), jnp.float32),
        pltpu.VMEM((_HID,), jnp.float32),
        pltpu.VMEM((_C, 16), jnp.float32),
        pltpu.VMEM((_C, 16), jnp.float32),
        pltpu.VMEM((_C + 8,), jnp.float32),
        pltpu.VMEM((_C + 8,), jnp.float32),
        pltpu.SemaphoreType.DMA,
        pltpu.SemaphoreType.DMA,
        pltpu.SemaphoreType.DMA,
        pltpu.SemaphoreType.DMA,
        pltpu.SemaphoreType.DMA,
        pltpu.SemaphoreType.DMA,
    ],
)(_sc_body)


def kernel(input_ids, position_ids, word_emb, pos_emb, ln_gamma, ln_beta):
    ids = input_ids.reshape(_N // _C, _C).astype(jnp.int32)
    pids = position_ids.reshape(_SRC // _PC, _PC).astype(jnp.int32)
    out = _sc_embed(ids, pids, word_emb, pos_emb, ln_gamma, ln_beta)
    return out.reshape(_SRC, _BATCH, _HID)


# speedup vs baseline: 1.2717x; 1.2717x over previous
import functools

import jax
import jax.numpy as jnp
from jax import lax
from jax.experimental import pallas as pl
from jax.experimental.pallas import tpu as pltpu
from jax.experimental.pallas import tpu_sc as plsc

_HID = 1024
_SRC = 2048
_BATCH = 4
_N = _SRC * _BATCH
_NW = 32
_RPW = _N // _NW
_C = 16
_NCH = _RPW // _C
_PC = _C // _BATCH
_NL = _HID // 16
_EPS = 1e-5
_UNROLL = 8
_NJB = _NL // _UNROLL


def _sc_body(ids_hbm, pids_hbm, wemb_hbm, pemb_hbm, gam_hbm, bet_hbm,
             out_hbm,
             ids_all, pid_all, tok_a, tok_b, pos_a, pos_b,
             out_a, out_b, g_v, b_v, sacc_v, qacc_v, m_v, r_v,
             sem_ta, sem_tb, sem_pa, sem_pb, sem_wa, sem_wb):
    wid = lax.axis_index("s") * 2 + lax.axis_index("c")
    lanes = lax.iota(jnp.int32, 16)
    z = jnp.zeros((16,), jnp.float32)

    pltpu.sync_copy(ids_hbm.at[pl.ds(wid * _NCH, _NCH), :], ids_all)
    pltpu.sync_copy(pids_hbm.at[pl.ds(wid * _NCH, _NCH), :], pid_all)
    pltpu.sync_copy(gam_hbm, g_v)
    pltpu.sync_copy(bet_hbm, b_v)

    def issue_gather(c, tok_v, pos_v, sem_t, sem_p):
        pltpu.async_copy(wemb_hbm.at[ids_all.at[c]], tok_v, sem_t)
        pltpu.async_copy(pemb_hbm.at[pid_all.at[c]], pos_v, sem_p)

    def drain_gather(tok_v, pos_v, sem_t, sem_p):
        pltpu.make_async_copy(wemb_hbm.at[pl.ds(0, _C), :], tok_v,
                              sem_t).wait()
        pltpu.make_async_copy(pemb_hbm.at[pl.ds(0, _PC), :], pos_v,
                              sem_p).wait()

    def drain_writeback(out_v, sem_w):
        pltpu.make_async_copy(out_v, out_hbm.at[pl.ds(0, _C), :],
                              sem_w).wait()

    def compute(c, tok_v, pos_v, out_v, sem_w):
        base = pl.multiple_of(wid * _RPW + c * _C, _C)

        @plsc.parallel_loop(0, _C)
        def pass1_row(i):
            p = i // _BATCH

            @plsc.parallel_loop(0, _NJB, unroll=4, carry=(z, z, z, z))
            def acc(jb, sc):
                s0, s1, q0, q1 = sc
                for u in range(_UNROLL):
                    off = jb * (_UNROLL * 16) + u * 16
                    x = (tok_v[i, pl.ds(off, 16)]
                         + pos_v[p, pl.ds(off, 16)])
                    out_v[i, pl.ds(off, 16)] = x
                    if u % 2 == 0:
                        s0 = s0 + x
                        q0 = q0 + x * x
                    else:
                        s1 = s1 + x
                        q1 = q1 + x * x
                return (s0, s1, q0, q1)

            s0, s1, q0, q1 = acc
            sacc_v[i, :] = s0 + s1
            qacc_v[i, :] = q0 + q1

        t = z
        tq = z
        for k in range(16):
            ck = jnp.full((16,), k, jnp.int32)
            t = t + plsc.load_gather(sacc_v, [lanes, ck])
            tq = tq + plsc.load_gather(qacc_v, [lanes, ck])
        mean16 = t * (1.0 / _HID)
        var16 = tq * (1.0 / _HID) - mean16 * mean16
        xv = var16 + _EPS
        ii = plsc.bitcast(xv, jnp.int32)
        ii = 0x5F3759DF - (ii >> 1)
        y = plsc.bitcast(ii, jnp.float32)
        y = y * (1.5 - 0.5 * xv * y * y)
        y = y * (1.5 - 0.5 * xv * y * y)
        y = y * (1.5 - 0.5 * xv * y * y)
        r_v[pl.ds(8, 16)] = y
        m_v[pl.ds(8, 16)] = mean16 * y

        for rb in range(_C // 4):
            yv = [plsc.load_gather(r_v, [jnp.full((16,), 8 + rb * 4 + i,
                                                  jnp.int32)])
                  for i in range(4)]
            nm = [plsc.load_gather(m_v, [jnp.full((16,), 8 + rb * 4 + i,
                                                  jnp.int32)])
                  for i in range(4)]

            @plsc.parallel_loop(0, _NJB, unroll=4)
            def norm(jb):
                for u in range(_UNROLL):
                    off = jb * (_UNROLL * 16) + u * 16
                    g = g_v[pl.ds(off, 16)]
                    b = b_v[pl.ds(off, 16)]
                    for i in range(4):
                        row = rb * 4 + i
                        x = out_v[row, pl.ds(off, 16)]
                        out_v[row, pl.ds(off, 16)] = (
                            (x * yv[i] - nm[i]) * g + b)
        pltpu.async_copy(out_v, out_hbm.at[pl.ds(base, _C), :], sem_w)

    issue_gather(0, tok_a, pos_a, sem_ta, sem_pa)

    def pair_body(g, carry):
        c0 = g * 2
        issue_gather(c0 + 1, tok_b, pos_b, sem_tb, sem_pb)
        drain_gather(tok_a, pos_a, sem_ta, sem_pa)

        @pl.when(g > 0)
        def _():
            drain_writeback(out_a, sem_wa)

        compute(c0, tok_a, pos_a, out_a, sem_wa)

        @pl.when(g < _NCH // 2 - 1)
        def _():
            issue_gather(c0 + 2, tok_a, pos_a, sem_ta, sem_pa)

        drain_gather(tok_b, pos_b, sem_tb, sem_pb)

        @pl.when(g > 0)
        def _():
            drain_writeback(out_b, sem_wb)

        compute(c0 + 1, tok_b, pos_b, out_b, sem_wb)
        return carry

    lax.fori_loop(0, _NCH // 2, pair_body, 0)
    drain_writeback(out_a, sem_wa)
    drain_writeback(out_b, sem_wb)


_sc_embed = functools.partial(
    pl.kernel,
    mesh=plsc.VectorSubcoreMesh(core_axis_name="c", subcore_axis_name="s"),
    out_type=jax.ShapeDtypeStruct((_N, _HID), jnp.float32),
    compiler_params=pltpu.CompilerParams(needs_layout_passes=False),
    scratch_types=[
        pltpu.VMEM((_NCH, _C), jnp.int32),
        pltpu.VMEM((_NCH, _PC), jnp.int32),
        pltpu.VMEM((_C, _HID), jnp.float32),
        pltpu.VMEM((_C, _HID), jnp.float32),
        pltpu.VMEM((_PC, _HID), jnp.float32),
        pltpu.VMEM((_PC, _HID), jnp.float32),
        pltpu.VMEM((_C, _HID), jnp.float32),
        pltpu.VMEM((_C, _HID), jnp.float32),
        pltpu.VMEM((_HID,), jnp.float32),
        pltpu.VMEM((_HID,), jnp.float32),
        pltpu.VMEM((_C, 16), jnp.float32),
        pltpu.VMEM((_C, 16), jnp.float32),
        pltpu.VMEM((_C + 8,), jnp.float32),
        pltpu.VMEM((_C + 8,), jnp.float32),
        pltpu.SemaphoreType.DMA,
        pltpu.SemaphoreType.DMA,
        pltpu.SemaphoreType.DMA,
        pltpu.SemaphoreType.DMA,
        pltpu.SemaphoreType.DMA,
        pltpu.SemaphoreType.DMA,
    ],
)(_sc_body)


def kernel(input_ids, position_ids, word_emb, pos_emb, ln_gamma, ln_beta):
    ids = input_ids.reshape(_N // _C, _C).astype(jnp.int32)
    pids = position_ids.reshape(_SRC // _PC, _PC).astype(jnp.int32)
    out = _sc_embed(ids, pids, word_emb, pos_emb, ln_gamma, ln_beta)
    return out.reshape(_SRC, _BATCH, _HID)
